# baseline (speedup 1.0000x reference)
import functools

import jax
import jax.numpy as jnp
from jax.experimental import pallas as pl
from jax.experimental.pallas import tpu as pltpu
from jax.experimental.pallas import tpu_sc as plsc

C_IN = 384
C_OUT = 96
W = 224
H = 224
WBLK = 32
NLANE = 16
NVREG = C_IN // NLANE


def _sc_rank_body(p_hbm, n_hbm, rank_hbm, p_v, n_v, v_v, res_v):
    cid = jax.lax.axis_index("c")
    sid = jax.lax.axis_index("s")

    @pl.when(cid == 0)
    def _rank():
        pltpu.sync_copy(p_hbm, p_v)
        pltpu.sync_copy(n_hbm, n_v)
        for t in range(NVREG):
            v_v[pl.ds(NLANE * t, NLANE)] = (
                p_v[pl.ds(NLANE * t, NLANE)] + n_v[pl.ds(NLANE * t, NLANE)])
        one = jnp.full((NLANE,), 1, jnp.int32)
        zero = jnp.zeros((NLANE,), jnp.int32)
        sid_b = zero + sid
        for k in range(NVREG):
            sel = v_v[pl.ds(NLANE * k, NLANE)]
            vi = sel.at[sid_b].get(mode="promise_in_bounds")
            i_b = sid_b + (NLANE * k)
            cnt = zero
            for t in range(NVREG):
                vj = v_v[pl.ds(NLANE * t, NLANE)]
                jl = jax.lax.iota(jnp.int32, NLANE) + (NLANE * t)
                gt = jnp.where(vj > vi, one, zero)
                eq = jnp.where(vj == vi, one, zero)
                lt = jnp.where(jl < i_b, one, zero)
                cnt = cnt + gt + eq * lt
            res_v[k] = cnt
        pltpu.sync_copy(res_v, rank_hbm.at[sid])


def _sc_ranks(params, noise):
    mesh = plsc.VectorSubcoreMesh(core_axis_name="c", subcore_axis_name="s")
    kern = functools.partial(
        pl.kernel,
        mesh=mesh,
        out_type=jax.ShapeDtypeStruct((NLANE, NVREG, NLANE), jnp.int32),
        scratch_types=[
            pltpu.VMEM((C_IN,), jnp.float32),
            pltpu.VMEM((C_IN,), jnp.float32),
            pltpu.VMEM((C_IN,), jnp.float32),
            pltpu.VMEM((NVREG, NLANE), jnp.int32),
        ],
    )(_sc_rank_body)
    return kern(params, noise)


def _body(p_ref, n_ref, rank_ref, x_ref, o_ref, g_s, m3t_s):
    bb = pl.program_id(0)
    cc = pl.program_id(1)

    @pl.when((bb == 0) & (cc == 0))
    def _prologue():
        vrow = (p_ref[...] + n_ref[...]).reshape(1, C_IN)
        vcol = jnp.transpose(vrow)
        rks = jnp.sum(rank_ref[...], axis=2)
        rr16 = jax.lax.broadcasted_iota(jnp.int32, (NLANE, C_OUT), 1)
        vals = jnp.zeros((1, C_OUT), jnp.float32)
        for k in range(NVREG):
            oh_k = (rks[:, k:k + 1] == rr16).astype(jnp.float32)
            g_s[NLANE * k:NLANE * (k + 1), :] = oh_k
            vcol_k = vcol[NLANE * k:NLANE * (k + 1), :]
            vals = vals + jnp.sum(oh_k * vcol_k, axis=0, keepdims=True)
        t3 = jnp.concatenate([vals, vals, vals], axis=1)
        pat = [t3[:, 0:H], t3[:, 32:32 + H], t3[:, 64:64 + H]]
        for r in range(3):
            blk = jnp.concatenate(
                [pat[r % 3], pat[(r + 2) % 3], pat[(r + 1) % 3]], axis=0)
            m3t_s[r] = jnp.tile(blk, (C_OUT // 3, 1))

    w0 = cc * WBLK
    for k in range(WBLK):
        r = jax.lax.rem(w0 + k, 3)
        xw = x_ref[0, k]
        z = jax.lax.dot_general(
            g_s[...], xw, (((0,), (1,)), ((), ())),
            preferred_element_type=jnp.float32)
        o_ref[0, :, k, :] = z * m3t_s[r]


def kernel(input, params, noise):
    ranks = _sc_ranks(params, noise)
    b = input.shape[0]
    xt = jnp.transpose(input, (0, 2, 3, 1))
    out = pl.pallas_call(
        _body,
        grid=(b, W // WBLK),
        in_specs=[
            pl.BlockSpec((C_IN,), lambda bb, cc: (0,)),
            pl.BlockSpec((C_IN,), lambda bb, cc: (0,)),
            pl.BlockSpec((NLANE, NVREG, NLANE), lambda bb, cc: (0, 0, 0)),
            pl.BlockSpec((1, WBLK, H, C_IN), lambda bb, cc: (bb, cc, 0, 0)),
        ],
        out_specs=pl.BlockSpec((1, C_OUT, WBLK, H), lambda bb, cc: (bb, 0, cc, 0)),
        out_shape=jax.ShapeDtypeStruct((b, C_OUT, W, H), jnp.float32),
        scratch_shapes=[
            pltpu.VMEM((C_IN, C_OUT), jnp.float32),
            pltpu.VMEM((3, C_OUT, H), jnp.float32),
        ],
    )(params, noise, ranks, xt)
    return out


# speedup vs baseline: 1.0830x; 1.0830x over previous
import functools

import jax
import jax.numpy as jnp
from jax.experimental import pallas as pl
from jax.experimental.pallas import tpu as pltpu
from jax.experimental.pallas import tpu_sc as plsc

C_IN = 384
C_OUT = 96
W = 224
H = 224
WBLK = 56
NLANE = 16
NVREG = C_IN // NLANE


def _sc_rank_body(p_hbm, n_hbm, rank_hbm, v_hbm, p_v, n_v, v_v, res_v, resv_v):
    cid = jax.lax.axis_index("c")
    sid = jax.lax.axis_index("s")

    @pl.when(cid == 0)
    def _rank():
        pltpu.sync_copy(p_hbm, p_v)
        pltpu.sync_copy(n_hbm, n_v)
        for t in range(NVREG):
            v_v[pl.ds(NLANE * t, NLANE)] = (
                p_v[pl.ds(NLANE * t, NLANE)] + n_v[pl.ds(NLANE * t, NLANE)])
        one = jnp.full((NLANE,), 1, jnp.int32)
        zero = jnp.zeros((NLANE,), jnp.int32)
        sid_b = zero + sid
        for k in range(NVREG):
            sel = v_v[pl.ds(NLANE * k, NLANE)]
            vi = sel.at[sid_b].get(mode="promise_in_bounds")
            i_b = sid_b + (NLANE * k)
            cnt = zero
            for t in range(NVREG):
                vj = v_v[pl.ds(NLANE * t, NLANE)]
                jl = jax.lax.iota(jnp.int32, NLANE) + (NLANE * t)
                gt = jnp.where(vj > vi, one, zero)
                eq = jnp.where(vj == vi, one, zero)
                lt = jnp.where(jl < i_b, one, zero)
                cnt = cnt + gt + eq * lt
            res_v[pl.ds(NLANE * k, NLANE)] = cnt
            resv_v[pl.ds(NLANE * k, NLANE)] = vi
        pltpu.sync_copy(res_v, rank_hbm.at[sid])
        pltpu.sync_copy(resv_v, v_hbm.at[sid])


def _sc_ranks(params, noise):
    mesh = plsc.VectorSubcoreMesh(core_axis_name="c", subcore_axis_name="s")
    kern = functools.partial(
        pl.kernel,
        mesh=mesh,
        out_type=[
            jax.ShapeDtypeStruct((NLANE, C_IN), jnp.int32),
            jax.ShapeDtypeStruct((NLANE, C_IN), jnp.float32),
        ],
        scratch_types=[
            pltpu.VMEM((C_IN,), jnp.float32),
            pltpu.VMEM((C_IN,), jnp.float32),
            pltpu.VMEM((C_IN,), jnp.float32),
            pltpu.VMEM((C_IN,), jnp.int32),
            pltpu.VMEM((C_IN,), jnp.float32),
        ],
    )(_sc_rank_body)
    return kern(params, noise)


def _body(rank_ref, v_ref, x_ref, o_ref, g_s, m3t_s):
    bb = pl.program_id(0)
    cc = pl.program_id(1)

    @pl.when((bb == 0) & (cc == 0))
    def _prologue():
        rr16 = jax.lax.broadcasted_iota(jnp.int32, (NLANE, C_OUT), 1)
        vals = jnp.zeros((1, C_OUT), jnp.float32)
        for k in range(NVREG):
            sl = slice(NLANE * k, NLANE * (k + 1))
            rank_k = jnp.sum(rank_ref[:, sl], axis=1, keepdims=True)
            oh_k = (rank_k == rr16).astype(jnp.float32)
            g_s[sl, :] = oh_k
            vcol_k = v_ref[:, NLANE * k:NLANE * k + 1]
            vals = vals + jnp.sum(oh_k * vcol_k, axis=0, keepdims=True)
        t3 = jnp.concatenate([vals, vals, vals], axis=1)
        pat = [t3[:, 0:H], t3[:, 32:32 + H], t3[:, 64:64 + H]]
        for r in range(3):
            blk = jnp.concatenate(
                [pat[r % 3], pat[(r + 2) % 3], pat[(r + 1) % 3]], axis=0)
            m3t_s[r] = jnp.tile(blk, (C_OUT // 3, 1))

    w0 = cc * WBLK
    for k in range(WBLK):
        r = jax.lax.rem(w0 + k, 3)
        xw = x_ref[0, k]
        z = jax.lax.dot_general(
            g_s[...], xw, (((0,), (1,)), ((), ())),
            preferred_element_type=jnp.float32)
        o_ref[0, :, k, :] = z * m3t_s[r]


def kernel(input, params, noise):
    ranks, vsc = _sc_ranks(params, noise)
    b = input.shape[0]
    xt = jnp.transpose(input, (0, 2, 3, 1))
    out = pl.pallas_call(
        _body,
        grid=(b, W // WBLK),
        in_specs=[
            pl.BlockSpec((NLANE, C_IN), lambda bb, cc: (0, 0)),
            pl.BlockSpec((NLANE, C_IN), lambda bb, cc: (0, 0)),
            pl.BlockSpec((1, WBLK, H, C_IN), lambda bb, cc: (bb, cc, 0, 0)),
        ],
        out_specs=pl.BlockSpec((1, C_OUT, WBLK, H), lambda bb, cc: (bb, 0, cc, 0)),
        out_shape=jax.ShapeDtypeStruct((b, C_OUT, W, H), jnp.float32),
        scratch_shapes=[
            pltpu.VMEM((C_IN, C_OUT), jnp.float32),
            pltpu.VMEM((3, C_OUT, H), jnp.float32),
        ],
    )(ranks, vsc, xt)
    return out
